# Initial kernel scaffold; baseline (speedup 1.0000x reference)
import functools

import jax
import jax.numpy as jnp
from jax import lax
from jax.experimental import pallas as pl
from jax.experimental.pallas import tpu as pltpu
from jax.experimental.pallas import tpu_sc as plsc

NC, NS = 2, 16
NW = NC * NS
CHUNK = 128
SUB = 8
GROUP = SUB * CHUNK


def _sc_gather(ids4, emb):
    _, G, _, _ = ids4.shape
    D = emb.shape[1]

    @functools.partial(
        pl.kernel,
        mesh=plsc.VectorSubcoreMesh(core_axis_name="c", subcore_axis_name="s"),
        out_type=jax.ShapeDtypeStruct((NW * G * GROUP, D), jnp.float32),
        scratch_types=[
            pltpu.VMEM((SUB, CHUNK), jnp.int32),
            pltpu.VMEM((GROUP, D), jnp.float32),
            pltpu.SemaphoreType.DMA,
        ],
    )
    def k(ids_hbm, emb_hbm, out_hbm, idx_v, rows_v, sem):
        wid = lax.axis_index("s") * NC + lax.axis_index("c")

        def body(g, carry):
            pltpu.sync_copy(ids_hbm.at[wid, g], idx_v)
            cps = [
                pltpu.async_copy(
                    emb_hbm.at[idx_v.at[j]],
                    rows_v.at[pl.ds(j * CHUNK, CHUNK)],
                    sem,
                )
                for j in range(SUB)
            ]
            for cp in cps:
                cp.wait()
            pltpu.sync_copy(
                rows_v, out_hbm.at[pl.ds((wid * G + g) * GROUP, GROUP)]
            )
            return carry

        lax.fori_loop(0, G, body, 0)

    return k(ids4, emb)


def _tc_linear(xp, w4, b4):
    M = xp.shape[0]
    BM = 2048

    def body(x_ref, w_ref, b_ref, o_ref):
        o_ref[...] = (
            jnp.dot(x_ref[...], w_ref[...], preferred_element_type=jnp.float32)
            + b_ref[...]
        )

    return pl.pallas_call(
        body,
        grid=(M // BM,),
        in_specs=[
            pl.BlockSpec((BM, 128), lambda i: (i, 0)),
            pl.BlockSpec((128, 128), lambda i: (0, 0)),
            pl.BlockSpec((1, 128), lambda i: (0, 0)),
        ],
        out_specs=pl.BlockSpec((BM, 128), lambda i: (i, 0)),
        out_shape=jax.ShapeDtypeStruct((M, 128), jnp.float32),
    )(xp, w4, b4)


def kernel(input_ids, emb, W, b):
    Bt, L = input_ids.shape
    V, D = emb.shape
    N = Bt * L
    G = N // (NW * GROUP)
    ids4 = input_ids.astype(jnp.int32).reshape(NW, G, SUB, CHUNK)
    x = _sc_gather(ids4, emb)
    w4 = jnp.kron(jnp.eye(4, dtype=W.dtype), W.T)
    b4 = jnp.tile(b, 4).reshape(1, 4 * D)
    y = _tc_linear(x.reshape(N // 4, 4 * D), w4, b4)
    return y.reshape(Bt, L, D)


# speedup vs baseline: 18.2499x; 18.2499x over previous
import functools

import jax
import jax.numpy as jnp
from jax import lax
from jax.experimental import pallas as pl
from jax.experimental.pallas import tpu as pltpu
from jax.experimental.pallas import tpu_sc as plsc

NC, NS = 2, 16
NW = NC * NS
CHUNK = 128
SUB = 8
GROUP = SUB * CHUNK


def _sc_gather(ids4, emb):
    _, G, _, _ = ids4.shape
    D = emb.shape[1]

    @functools.partial(
        pl.kernel,
        mesh=plsc.VectorSubcoreMesh(core_axis_name="c", subcore_axis_name="s"),
        out_type=jax.ShapeDtypeStruct((NW * G * GROUP, D), jnp.float32),
        scratch_types=[
            pltpu.VMEM((SUB, CHUNK), jnp.int32),
            pltpu.VMEM((GROUP, D), jnp.float32),
            pltpu.SemaphoreType.DMA,
        ],
        compiler_params=pltpu.CompilerParams(use_tc_tiling_on_sc=False),
    )
    def k(ids_hbm, emb_hbm, out_hbm, idx_v, rows_v, sem):
        wid = lax.axis_index("s") * NC + lax.axis_index("c")

        def body(g, carry):
            pltpu.sync_copy(ids_hbm.at[wid, g], idx_v)
            cps = [
                pltpu.async_copy(
                    emb_hbm.at[idx_v.at[j]],
                    rows_v.at[pl.ds(j * CHUNK, CHUNK)],
                    sem,
                )
                for j in range(SUB)
            ]
            for cp in cps:
                cp.wait()
            pltpu.sync_copy(
                rows_v, out_hbm.at[pl.ds((wid * G + g) * GROUP, GROUP)]
            )
            return carry

        lax.fori_loop(0, G, body, 0)

    return k(ids4, emb)


def _tc_linear(xp, w4, b4):
    M = xp.shape[0]
    BM = 2048

    def body(x_ref, w_ref, b_ref, o_ref):
        o_ref[...] = (
            jnp.dot(x_ref[...], w_ref[...], preferred_element_type=jnp.float32)
            + b_ref[...]
        )

    return pl.pallas_call(
        body,
        grid=(M // BM,),
        in_specs=[
            pl.BlockSpec((BM, 128), lambda i: (i, 0)),
            pl.BlockSpec((128, 128), lambda i: (0, 0)),
            pl.BlockSpec((1, 128), lambda i: (0, 0)),
        ],
        out_specs=pl.BlockSpec((BM, 128), lambda i: (i, 0)),
        out_shape=jax.ShapeDtypeStruct((M, 128), jnp.float32),
    )(xp, w4, b4)


def kernel(input_ids, emb, W, b):
    Bt, L = input_ids.shape
    V, D = emb.shape
    N = Bt * L
    G = N // (NW * GROUP)
    ids4 = input_ids.astype(jnp.int32).reshape(NW, G, SUB, CHUNK)
    x = _sc_gather(ids4, emb)
    w4 = jnp.kron(jnp.eye(4, dtype=W.dtype), W.T)
    b4 = jnp.tile(b, 4).reshape(1, 4 * D)
    y = _tc_linear(x.reshape(N // 4, 4 * D), w4, b4)
    return y.reshape(Bt, L, D)
